# baseline (speedup 1.0000x reference)
import functools
import math

import jax
import jax.numpy as jnp
from jax.experimental import pallas as pl
from jax.experimental.pallas import tpu as pltpu

_NUM_UNITS = 1024
_SCALE = float(_NUM_UNITS) ** 0.5
_BT = 256


def _pe_tile(o_ref, sinx_ref, cosx_ref, val_ref, *, bt, num_units):
    t = pl.program_id(0)
    i = pl.program_id(1)

    @pl.when(i == 0)
    def _():
        col = jax.lax.broadcasted_iota(jnp.int32, (1, num_units), 1)
        inv = jnp.exp(
            col.astype(jnp.float32)
            * (-2.0 * math.log(10000.0) / float(num_units))
        )

        @pl.when(t == 0)
        def _():
            phase = (col % 2).astype(jnp.float32) * (math.pi / 2.0)
            r8 = jax.lax.broadcasted_iota(
                jnp.int32, (8, num_units), 0
            ).astype(jnp.float32)
            x8 = r8 * inv + phase
            s = jnp.sin(x8)
            c = jnp.cos(x8)
            k = 8
            while k < bt:
                sd = jnp.sin(float(k) * inv)
                cd = jnp.cos(float(k) * inv)
                s, c = (
                    jnp.concatenate([s, s * cd + c * sd], axis=0),
                    jnp.concatenate([c, c * cd - s * sd], axis=0),
                )
                k *= 2
            sinx_ref[...] = s
            cosx_ref[...] = c

        p = (t * bt).astype(jnp.float32) * inv
        sp = jnp.sin(p) * _SCALE
        cp = jnp.cos(p) * _SCALE
        val = sp * cosx_ref[...] + cp * sinx_ref[...]
        row = jax.lax.broadcasted_iota(jnp.int32, (bt, num_units), 0)
        val = jnp.where((row + t * bt) == 0, 0.0, val)
        val_ref[...] = val

    o_ref[...] = val_ref[...][None]


def kernel(inputs):
    n, t_len = inputs.shape
    num_units = _NUM_UNITS
    bt = _BT
    grid = (t_len // bt, n)
    out = pl.pallas_call(
        functools.partial(_pe_tile, bt=bt, num_units=num_units),
        grid=grid,
        out_specs=pl.BlockSpec((1, bt, num_units), lambda t, i: (i, t, 0)),
        out_shape=jax.ShapeDtypeStruct((n, t_len, num_units), jnp.float32),
        scratch_shapes=[
            pltpu.VMEM((bt, num_units), jnp.float32),
            pltpu.VMEM((bt, num_units), jnp.float32),
            pltpu.VMEM((bt, num_units), jnp.float32),
        ],
    )()
    return out


# speedup vs baseline: 1.7826x; 1.7826x over previous
import functools
import math

import jax
import jax.numpy as jnp
from jax.experimental import pallas as pl
from jax.experimental.pallas import tpu as pltpu

_NUM_UNITS = 1024
_SCALE = float(_NUM_UNITS) ** 0.5
_BT = 256


def _pe_tile(o_ref, sinx_ref, cosx_ref, *, bt, num_units):
    t = pl.program_id(0)
    col = jax.lax.broadcasted_iota(jnp.int32, (1, num_units), 1)
    inv = jnp.exp(
        col.astype(jnp.float32) * (-2.0 * math.log(10000.0) / float(num_units))
    )

    @pl.when(t == 0)
    def _():
        phase = (col % 2).astype(jnp.float32) * (math.pi / 2.0)
        r8 = jax.lax.broadcasted_iota(jnp.int32, (8, num_units), 0).astype(
            jnp.float32
        )
        x8 = r8 * inv + phase
        s = jnp.sin(x8)
        c = jnp.cos(x8)
        k = 8
        while k < bt:
            sd = jnp.sin(float(k) * inv)
            cd = jnp.cos(float(k) * inv)
            s, c = (
                jnp.concatenate([s, s * cd + c * sd], axis=0),
                jnp.concatenate([c, c * cd - s * sd], axis=0),
            )
            k *= 2
        sinx_ref[...] = s
        cosx_ref[...] = c

    p = (t * bt).astype(jnp.float32) * inv
    sp = jnp.sin(p) * _SCALE
    cp = jnp.cos(p) * _SCALE
    val = sp * cosx_ref[...] + cp * sinx_ref[...]
    o_ref[...] = jnp.broadcast_to(val[None], o_ref.shape)

    @pl.when(t == 0)
    def _():
        o_ref[:, 0:1, :] = jnp.zeros_like(o_ref[:, 0:1, :])


def kernel(inputs):
    n, t_len = inputs.shape
    num_units = _NUM_UNITS
    bt = _BT
    grid = (t_len // bt,)
    out = pl.pallas_call(
        functools.partial(_pe_tile, bt=bt, num_units=num_units),
        grid=grid,
        out_specs=pl.BlockSpec((n, bt, num_units), lambda g: (0, g, 0)),
        out_shape=jax.ShapeDtypeStruct((n, t_len, num_units), jnp.float32),
        scratch_shapes=[
            pltpu.VMEM((bt, num_units), jnp.float32),
            pltpu.VMEM((bt, num_units), jnp.float32),
        ],
    )()
    return out
